# baseline (speedup 1.0000x reference)
import functools

import jax
import jax.numpy as jnp
from jax import lax
from jax.experimental import pallas as pl
from jax.experimental.pallas import tpu as pltpu
from jax.experimental.pallas import tpu_sc as plsc

N = 10000
D = 128
E = 320000

NC = 2
NS = 16
NW = NC * NS

NPAD = 10240
ROWS_PER_TILE = NPAD // NS

CH = 125
CPT = 80
PCH = 40

_mesh = plsc.VectorSubcoreMesh(
    core_axis_name="c", subcore_axis_name="s", num_cores=NC, num_subcores=NS
)


@functools.partial(
    pl.kernel,
    out_type=jax.ShapeDtypeStruct((NC, NPAD), jnp.float32),
    mesh=_mesh,
    scratch_types=[
        pltpu.VMEM((CPT, CH), jnp.int32),
        pltpu.VMEM((CH,), jnp.float32),
        pltpu.VMEM_SHARED((NPAD,), jnp.float32),
        pltpu.SemaphoreType.DMA,
    ],
)
def _deg_kernel(dst_hbm, ones_hbm, zeros_hbm, out_hbm, idx_v, ones_v, deg_sh,
                sem):
    c = lax.axis_index("c")
    s = lax.axis_index("s")
    wid = s * NC + c
    row0 = s * ROWS_PER_TILE
    pltpu.sync_copy(zeros_hbm, deg_sh.at[pl.ds(row0, ROWS_PER_TILE)])
    pltpu.sync_copy(ones_hbm, ones_v)
    pltpu.sync_copy(dst_hbm.at[wid], idx_v)
    plsc.subcore_barrier()

    def fire(j, carry):
        pltpu.async_copy(ones_v, deg_sh.at[idx_v.at[j]], sem, add=True)
        return carry

    lax.fori_loop(0, CPT, fire, 0)

    def drain(j, carry):
        pltpu.make_async_copy(ones_v, deg_sh.at[idx_v.at[j]], sem).wait()
        return carry

    lax.fori_loop(0, CPT, drain, 0)
    plsc.subcore_barrier()
    pltpu.sync_copy(deg_sh.at[pl.ds(row0, ROWS_PER_TILE)],
                    out_hbm.at[c, pl.ds(row0, ROWS_PER_TILE)])


@functools.partial(
    pl.kernel,
    out_type=jax.ShapeDtypeStruct((NC, NPAD, D), jnp.float32),
    mesh=_mesh,
    scratch_types=[
        pltpu.VMEM((PCH, CH), jnp.int32),
        pltpu.VMEM((PCH, CH), jnp.int32),
        pltpu.VMEM((CH, D), jnp.float32),
        pltpu.VMEM((CH, D), jnp.float32),
        pltpu.VMEM_SHARED((NPAD, D), jnp.float32),
        pltpu.SemaphoreType.DMA,
        pltpu.SemaphoreType.DMA,
    ],
)
def _edge_kernel(g_hbm, src_hbm, dst_hbm, zeros_hbm, out_hbm,
                 src_v, dst_v, buf0_v, buf1_v, acc_sh, sem0, sem1):
    c = lax.axis_index("c")
    s = lax.axis_index("s")
    wid = s * NC + c
    row0 = s * ROWS_PER_TILE

    bufs = (buf0_v, buf1_v)
    sems = (sem0, sem1)

    def gather(j, b):
        pltpu.async_copy(g_hbm.at[src_v.at[j]], bufs[b], sems[b])

    def drain_scatter(j, b):
        pltpu.make_async_copy(g_hbm.at[src_v.at[j]], bufs[b], sems[b]).wait()
        pltpu.sync_copy(bufs[b], acc_sh.at[dst_v.at[j]], add=True)

    def load_idx(p):
        pltpu.sync_copy(src_hbm.at[wid, pl.ds(p * PCH, PCH)], src_v)
        pltpu.sync_copy(dst_hbm.at[wid, pl.ds(p * PCH, PCH)], dst_v)

    def run_phase():
        gather(0, 0)

        def body(i, carry):
            j = i * 2
            gather(j + 1, 1)
            drain_scatter(j, 0)

            @pl.when(j + 2 < PCH)
            def _():
                gather(j + 2, 0)

            drain_scatter(j + 1, 1)
            return carry

        lax.fori_loop(0, PCH // 2, body, 0)

    load_idx(0)
    pltpu.sync_copy(zeros_hbm, acc_sh.at[pl.ds(row0, ROWS_PER_TILE)])
    plsc.subcore_barrier()
    run_phase()
    load_idx(1)
    run_phase()
    plsc.subcore_barrier()
    pltpu.sync_copy(acc_sh.at[pl.ds(row0, ROWS_PER_TILE)],
                    out_hbm.at[c, pl.ds(row0, ROWS_PER_TILE)])


_BR = 256


def _dinv_block(deg_ref):
    i = pl.program_id(0)
    sl = pl.ds(i * _BR, _BR)
    deg = deg_ref[0, sl] + deg_ref[1, sl] + 1.0
    return lax.rsqrt(jnp.maximum(deg, 1e-12))


def _mm_body(x_ref, wt_ref, h_ref):
    h_ref[...] = jnp.dot(x_ref[...], wt_ref[...],
                         preferred_element_type=jnp.float32)


def _scale_body(deg_ref, h_ref, g_ref):
    dinv = _dinv_block(deg_ref)
    g_ref[...] = h_ref[...] * dinv[:, None]


def _fin_body(w_ref, deg_ref, acc_ref, g_ref, b_ref, o_ref):
    dinv = _dinv_block(deg_ref)
    z = (acc_ref[0] + acc_ref[1] + g_ref[...]) * dinv[:, None] + b_ref[...]
    o_ref[...] = jnp.where(z >= 0, z, w_ref[0] * z)


def kernel(x, edge_index, W, b, prelu_weight):
    ei = edge_index.astype(jnp.int32).reshape(2, NW, CPT, CH)
    src_t = ei[0]
    dst_t = ei[1]
    wt = W.T

    ones1 = jnp.ones((CH,), jnp.float32)
    zeros1 = jnp.zeros((ROWS_PER_TILE,), jnp.float32)
    zerosd = jnp.zeros((ROWS_PER_TILE, D), jnp.float32)

    degp = _deg_kernel(dst_t, ones1, zeros1)

    h = pl.pallas_call(
        _mm_body,
        grid=(pl.cdiv(N, _BR),),
        in_specs=[
            pl.BlockSpec((_BR, D), lambda i: (i, 0)),
            pl.BlockSpec((D, D), lambda i: (0, 0)),
        ],
        out_specs=pl.BlockSpec((_BR, D), lambda i: (i, 0)),
        out_shape=jax.ShapeDtypeStruct((N, D), jnp.float32),
    )(x, wt)

    g = pl.pallas_call(
        _scale_body,
        grid=(pl.cdiv(N, _BR),),
        in_specs=[
            pl.BlockSpec((NC, NPAD), lambda i: (0, 0)),
            pl.BlockSpec((_BR, D), lambda i: (i, 0)),
        ],
        out_specs=pl.BlockSpec((_BR, D), lambda i: (i, 0)),
        out_shape=jax.ShapeDtypeStruct((N, D), jnp.float32),
    )(degp, h)

    accp = _edge_kernel(g, src_t, dst_t, zerosd)

    out = pl.pallas_call(
        _fin_body,
        grid=(pl.cdiv(N, _BR),),
        in_specs=[
            pl.BlockSpec(memory_space=pltpu.SMEM),
            pl.BlockSpec((NC, NPAD), lambda i: (0, 0)),
            pl.BlockSpec((NC, _BR, D), lambda i: (0, i, 0)),
            pl.BlockSpec((_BR, D), lambda i: (i, 0)),
            pl.BlockSpec((1, D), lambda i: (0, 0)),
        ],
        out_specs=pl.BlockSpec((_BR, D), lambda i: (i, 0)),
        out_shape=jax.ShapeDtypeStruct((N, D), jnp.float32),
    )(prelu_weight.reshape(1), degp, accp, g, b.reshape(1, D))

    return out


# speedup vs baseline: 1.1096x; 1.1096x over previous
import functools

import jax
import jax.numpy as jnp
from jax import lax
from jax.experimental import pallas as pl
from jax.experimental.pallas import tpu as pltpu
from jax.experimental.pallas import tpu_sc as plsc

N = 10000
D = 128
E = 320000

NC = 2
NS = 16
NW = NC * NS

NPAD = 10240
ROWS_PER_TILE = NPAD // NS

CH = 125
CPT = 80
PCH = 40

_mesh = plsc.VectorSubcoreMesh(
    core_axis_name="c", subcore_axis_name="s", num_cores=NC, num_subcores=NS
)


@functools.partial(
    pl.kernel,
    out_type=jax.ShapeDtypeStruct((NC, NPAD), jnp.float32),
    mesh=_mesh,
    scratch_types=[
        pltpu.VMEM((CPT, CH), jnp.int32),
        pltpu.VMEM((CH,), jnp.float32),
        pltpu.VMEM_SHARED((NPAD,), jnp.float32),
        pltpu.SemaphoreType.DMA,
    ],
)
def _deg_kernel(dst_hbm, ones_hbm, zeros_hbm, out_hbm, idx_v, ones_v, deg_sh,
                sem):
    c = lax.axis_index("c")
    s = lax.axis_index("s")
    wid = s * NC + c
    row0 = s * ROWS_PER_TILE
    pltpu.sync_copy(zeros_hbm, deg_sh.at[pl.ds(row0, ROWS_PER_TILE)])
    pltpu.sync_copy(ones_hbm, ones_v)
    pltpu.sync_copy(dst_hbm.at[wid], idx_v)
    plsc.subcore_barrier()

    def fire(j, carry):
        pltpu.async_copy(ones_v, deg_sh.at[idx_v.at[j]], sem, add=True)
        return carry

    lax.fori_loop(0, CPT, fire, 0)

    def drain(j, carry):
        pltpu.make_async_copy(ones_v, deg_sh.at[idx_v.at[j]], sem).wait()
        return carry

    lax.fori_loop(0, CPT, drain, 0)
    plsc.subcore_barrier()
    pltpu.sync_copy(deg_sh.at[pl.ds(row0, ROWS_PER_TILE)],
                    out_hbm.at[c, pl.ds(row0, ROWS_PER_TILE)])


@functools.partial(
    pl.kernel,
    out_type=jax.ShapeDtypeStruct((NC, NPAD, D), jnp.float32),
    mesh=_mesh,
    scratch_types=[
        pltpu.VMEM((PCH, CH), jnp.int32),
        pltpu.VMEM((PCH, CH), jnp.int32),
        pltpu.VMEM((CH, D), jnp.float32),
        pltpu.VMEM((CH, D), jnp.float32),
        pltpu.VMEM_SHARED((NPAD, D), jnp.float32),
        pltpu.SemaphoreType.DMA,
        pltpu.SemaphoreType.DMA,
    ],
)
def _edge_kernel(g_hbm, src_hbm, dst_hbm, zeros_hbm, out_hbm,
                 src_v, dst_v, buf0_v, buf1_v, acc_sh, sem0, sem1):
    c = lax.axis_index("c")
    s = lax.axis_index("s")
    wid = s * NC + c
    row0 = s * ROWS_PER_TILE

    bufs = (buf0_v, buf1_v)
    sems = (sem0, sem1)

    def gather(j, b):
        pltpu.async_copy(g_hbm.at[src_v.at[j]], bufs[b], sems[b])

    def drain_scatter(j, b):
        pltpu.make_async_copy(g_hbm.at[src_v.at[j]], bufs[b], sems[b]).wait()
        pltpu.sync_copy(bufs[b], acc_sh.at[dst_v.at[j]], add=True)

    def load_idx(p):
        pltpu.sync_copy(src_hbm.at[wid, pl.ds(p * PCH, PCH)], src_v)
        pltpu.sync_copy(dst_hbm.at[wid, pl.ds(p * PCH, PCH)], dst_v)

    def run_phase():
        gather(0, 0)

        def body(i, carry):
            j = i * 2
            gather(j + 1, 1)
            drain_scatter(j, 0)

            @pl.when(j + 2 < PCH)
            def _():
                gather(j + 2, 0)

            drain_scatter(j + 1, 1)
            return carry

        lax.fori_loop(0, PCH // 2, body, 0)

    load_idx(0)
    pltpu.sync_copy(zeros_hbm, acc_sh.at[pl.ds(row0, ROWS_PER_TILE)])
    plsc.subcore_barrier()
    run_phase()
    load_idx(1)
    run_phase()
    plsc.subcore_barrier()
    pltpu.sync_copy(acc_sh.at[pl.ds(row0, ROWS_PER_TILE)],
                    out_hbm.at[c, pl.ds(row0, ROWS_PER_TILE)])


_BR = 256


def _dinv_block(deg_ref):
    i = pl.program_id(0)
    sl = pl.ds(i * _BR, _BR)
    deg = deg_ref[0, sl] + deg_ref[1, sl] + 1.0
    return lax.rsqrt(jnp.maximum(deg, 1e-12))


def _mm_body(deg_ref, x_ref, wt_ref, g_ref):
    dinv = _dinv_block(deg_ref)
    h = jnp.dot(x_ref[...], wt_ref[...], preferred_element_type=jnp.float32)
    g_ref[...] = h * dinv[:, None]


def _fin_body(w_ref, deg_ref, acc_ref, g_ref, b_ref, o_ref):
    dinv = _dinv_block(deg_ref)
    z = (acc_ref[0] + acc_ref[1] + g_ref[...]) * dinv[:, None] + b_ref[...]
    o_ref[...] = jnp.where(z >= 0, z, w_ref[0] * z)


def kernel(x, edge_index, W, b, prelu_weight):
    ei = edge_index.astype(jnp.int32).reshape(2, NW, CPT, CH)
    src_t = ei[0]
    dst_t = ei[1]
    wt = W.T

    ones1 = jnp.ones((CH,), jnp.float32)
    zeros1 = jnp.zeros((ROWS_PER_TILE,), jnp.float32)
    zerosd = jnp.zeros((ROWS_PER_TILE, D), jnp.float32)

    degp = _deg_kernel(dst_t, ones1, zeros1)

    g = pl.pallas_call(
        _mm_body,
        grid=(pl.cdiv(N, _BR),),
        in_specs=[
            pl.BlockSpec((NC, NPAD), lambda i: (0, 0)),
            pl.BlockSpec((_BR, D), lambda i: (i, 0)),
            pl.BlockSpec((D, D), lambda i: (0, 0)),
        ],
        out_specs=pl.BlockSpec((_BR, D), lambda i: (i, 0)),
        out_shape=jax.ShapeDtypeStruct((N, D), jnp.float32),
    )(degp, x, wt)

    accp = _edge_kernel(g, src_t, dst_t, zerosd)

    out = pl.pallas_call(
        _fin_body,
        grid=(pl.cdiv(N, _BR),),
        in_specs=[
            pl.BlockSpec(memory_space=pltpu.SMEM),
            pl.BlockSpec((NC, NPAD), lambda i: (0, 0)),
            pl.BlockSpec((NC, _BR, D), lambda i: (0, i, 0)),
            pl.BlockSpec((_BR, D), lambda i: (i, 0)),
            pl.BlockSpec((1, D), lambda i: (0, 0)),
        ],
        out_specs=pl.BlockSpec((_BR, D), lambda i: (i, 0)),
        out_shape=jax.ShapeDtypeStruct((N, D), jnp.float32),
    )(prelu_weight.reshape(1), degp, accp, g, b.reshape(1, D))

    return out
